# Initial kernel scaffold; baseline (speedup 1.0000x reference)
import functools

import jax
import jax.numpy as jnp
from jax import lax
from jax.experimental import pallas as pl
from jax.experimental.pallas import tpu as pltpu
from jax.experimental.pallas import tpu_sc as plsc

N = 10000
E = 320000
D_NODE = 128
D_EDGE = 16
D_OUT = 128

NC = 2
NS = 16
NW = NC * NS
C = 128
E_PAD = 327680
T = E_PAD // (NW * C)
N_PAD = 10016
NPS = N_PAD // NS

_mesh = plsc.VectorSubcoreMesh(core_axis_name="c", subcore_axis_name="s")


def _x1_body(node_ref, w_ref, b_ref, o_ref):
    o_ref[...] = (
        jnp.dot(node_ref[...], w_ref[...], preferred_element_type=jnp.float32)
        + b_ref[...]
    )


def _x1(node, w1n_t, b_msg):
    return pl.pallas_call(
        _x1_body,
        out_shape=jax.ShapeDtypeStruct((N, D_OUT), jnp.float32),
    )(node, w1n_t, b_msg.reshape(1, D_OUT))


@functools.partial(
    pl.kernel,
    out_type=jax.ShapeDtypeStruct((E_PAD, D_OUT), jnp.float32),
    mesh=_mesh,
    scratch_types=[
        pltpu.VMEM((T, C), jnp.int32),
        pltpu.VMEM((C, D_OUT), jnp.float32),
        pltpu.SemaphoreType.DMA,
    ],
)
def _gather_k(x1_hbm, src_hbm, out_hbm, idx_v, rows_v, sem):
    cid = lax.axis_index("c")
    sid = lax.axis_index("s")
    wid = cid * NS + sid
    base = wid * (T * C)
    pltpu.sync_copy(src_hbm.at[wid], idx_v)

    def step(t, carry):
        pltpu.async_copy(x1_hbm.at[idx_v.at[t]], rows_v, sem).wait()
        pltpu.sync_copy(rows_v, out_hbm.at[pl.ds(base + t * C, C)])
        return carry

    lax.fori_loop(0, T, step, 0)


def _mlp_body(g_ref, ef_ref, w1e_ref, w2_ref, b2_ref, o_ref):
    m1 = jnp.maximum(
        g_ref[...]
        + jnp.dot(ef_ref[...], w1e_ref[...], preferred_element_type=jnp.float32),
        0.0,
    )
    o_ref[...] = jnp.maximum(
        jnp.dot(m1, w2_ref[...], preferred_element_type=jnp.float32) + b2_ref[...],
        0.0,
    )


def _mlp(g, ef_pad, w1e_t, w2_t, b2):
    BE = 5120
    grid = (E_PAD // BE,)
    return pl.pallas_call(
        _mlp_body,
        grid=grid,
        in_specs=[
            pl.BlockSpec((BE, D_OUT), lambda i: (i, 0)),
            pl.BlockSpec((BE, D_EDGE), lambda i: (i, 0)),
            pl.BlockSpec((D_EDGE, D_OUT), lambda i: (0, 0)),
            pl.BlockSpec((D_OUT, D_OUT), lambda i: (0, 0)),
            pl.BlockSpec((1, D_OUT), lambda i: (0, 0)),
        ],
        out_specs=pl.BlockSpec((BE, D_OUT), lambda i: (i, 0)),
        out_shape=jax.ShapeDtypeStruct((E_PAD, D_OUT), jnp.float32),
    )(g, ef_pad, w1e_t, w2_t, b2.reshape(1, D_OUT))


@functools.partial(
    pl.kernel,
    out_type=(
        jax.ShapeDtypeStruct((NC, N_PAD, D_OUT), jnp.float32),
        jax.ShapeDtypeStruct((NC, N_PAD), jnp.float32),
    ),
    mesh=_mesh,
    scratch_types=[
        pltpu.VMEM((T, C), jnp.int32),
        pltpu.VMEM((C, D_OUT), jnp.float32),
        pltpu.VMEM((C,), jnp.float32),
        pltpu.VMEM_SHARED((N_PAD, D_OUT), jnp.float32),
        pltpu.VMEM_SHARED((N_PAD,), jnp.float32),
        pltpu.SemaphoreType.DMA,
    ],
)
def _scatter_k(m_hbm, dst_hbm, z2_hbm, z1_hbm, s_out, d_out,
               dst_v, mrow_v, ones_v, acc_sh, deg_sh, sem):
    cid = lax.axis_index("c")
    sid = lax.axis_index("s")
    wid = cid * NS + sid
    base = wid * (T * C)

    pltpu.sync_copy(z2_hbm.at[pl.ds(sid * NPS, NPS)],
                    acc_sh.at[pl.ds(sid * NPS, NPS)])
    pltpu.sync_copy(z1_hbm.at[pl.ds(sid * NPS, NPS)],
                    deg_sh.at[pl.ds(sid * NPS, NPS)])
    for i in range(C // 16):
        ones_v[pl.ds(i * 16, 16)] = jnp.ones((16,), jnp.float32)
    pltpu.sync_copy(dst_hbm.at[wid], dst_v)
    plsc.subcore_barrier()

    def step(t, carry):
        pltpu.sync_copy(m_hbm.at[pl.ds(base + t * C, C)], mrow_v)
        pltpu.sync_copy(mrow_v, acc_sh.at[dst_v.at[t]], add=True)
        pltpu.sync_copy(ones_v, deg_sh.at[dst_v.at[t]], add=True)
        return carry

    lax.fori_loop(0, T, step, 0)
    plsc.subcore_barrier()

    pltpu.sync_copy(acc_sh.at[pl.ds(sid * NPS, NPS)],
                    s_out.at[cid, pl.ds(sid * NPS, NPS)])
    pltpu.sync_copy(deg_sh.at[pl.ds(sid * NPS, NPS)],
                    d_out.at[cid, pl.ds(sid * NPS, NPS)])


def _apply_body(node_ref, s_ref, d_ref, wan_ref, wah_ref, b_ref, o_ref):
    s = s_ref[0] + s_ref[1]
    deg = d_ref[0] + d_ref[1]
    inv = 1.0 / jnp.maximum(deg, 1.0)
    h = s * inv[:, None]
    o_ref[...] = (
        jnp.dot(node_ref[...], wan_ref[...], preferred_element_type=jnp.float32)
        + jnp.dot(h, wah_ref[...], preferred_element_type=jnp.float32)
        + b_ref[...]
    )


def _apply(node, s2, d2, wan_t, wah_t, b_apply):
    BN = 2000
    grid = (N // BN,)
    return pl.pallas_call(
        _apply_body,
        grid=grid,
        in_specs=[
            pl.BlockSpec((BN, D_NODE), lambda i: (i, 0)),
            pl.BlockSpec((NC, BN, D_OUT), lambda i: (0, i, 0)),
            pl.BlockSpec((NC, BN), lambda i: (0, i)),
            pl.BlockSpec((D_NODE, D_OUT), lambda i: (0, 0)),
            pl.BlockSpec((D_OUT, D_OUT), lambda i: (0, 0)),
            pl.BlockSpec((1, D_OUT), lambda i: (0, 0)),
        ],
        out_specs=pl.BlockSpec((BN, D_OUT), lambda i: (i, 0)),
        out_shape=jax.ShapeDtypeStruct((N, D_OUT), jnp.float32),
    )(node, s2, d2, wan_t, wah_t, b_apply.reshape(1, D_OUT))


def kernel(node_features, edge_index, edge_features,
           W_msg, b_msg, W_msg_2, b_msg_2, W_apply, b_apply):
    src = edge_index[0]
    dst = edge_index[1]

    w1n_t = W_msg[:, :D_NODE].T
    w1e_t = W_msg[:, D_NODE:].T
    w2_t = W_msg_2.T
    wan_t = W_apply[:, :D_NODE].T
    wah_t = W_apply[:, D_NODE:].T

    pad = E_PAD - E
    src_p = jnp.concatenate([src, jnp.zeros((pad,), jnp.int32)])
    dst_p = jnp.concatenate([dst, jnp.full((pad,), N, jnp.int32)])
    ef_p = jnp.concatenate(
        [edge_features, jnp.zeros((pad, D_EDGE), jnp.float32)]
    )
    src_r = src_p.reshape(NW, T, C)
    dst_r = dst_p.reshape(NW, T, C)

    x1 = _x1(node_features, w1n_t, b_msg)
    g = _gather_k(x1, src_r)
    m = _mlp(g, ef_p, w1e_t, w2_t, b_msg_2)
    z2 = jnp.zeros((N_PAD, D_OUT), jnp.float32)
    z1 = jnp.zeros((N_PAD,), jnp.float32)
    s2, d2 = _scatter_k(m, dst_r, z2, z1)
    s2 = s2[:, :N]
    d2 = d2[:, :N]
    return _apply(node_features, s2, d2, wan_t, wah_t, b_apply)


# speedup vs baseline: 2.2312x; 2.2312x over previous
import functools

import jax
import jax.numpy as jnp
from jax import lax
from jax.experimental import pallas as pl
from jax.experimental.pallas import tpu as pltpu
from jax.experimental.pallas import tpu_sc as plsc

N = 10000
E = 320000
D_NODE = 128
D_EDGE = 16
D_OUT = 128

NC = 2
NS = 16
NW = NC * NS
C = 128
E_PAD = 327680
T = E_PAD // (NW * C)
N_PAD = 10240
NPS = N_PAD // NS

@functools.lru_cache(maxsize=None)
def _sc_mesh():
    return plsc.VectorSubcoreMesh(
        core_axis_name="c", subcore_axis_name="s", num_cores=NC, num_subcores=NS
    )


def _x1_body(node_ref, w_ref, b_ref, o_ref):
    o_ref[...] = (
        jnp.dot(node_ref[...], w_ref[...], preferred_element_type=jnp.float32)
        + b_ref[...]
    )


def _x1(node, w1n_t, b_msg):
    return pl.pallas_call(
        _x1_body,
        out_shape=jax.ShapeDtypeStruct((N, D_OUT), jnp.float32),
    )(node, w1n_t, b_msg.reshape(1, D_OUT))


@functools.lru_cache(maxsize=None)
def _gather_kernel():
    return pl.kernel(
        _gather_body,
        out_type=jax.ShapeDtypeStruct((E_PAD, D_OUT), jnp.float32),
        mesh=_sc_mesh(),
        scratch_types=[
            pltpu.VMEM((T, C), jnp.int32),
            pltpu.VMEM((C, D_OUT), jnp.float32),
            pltpu.SemaphoreType.DMA,
        ],
    )


def _gather_body(x1_hbm, src_hbm, out_hbm, idx_v, rows_v, sem):
    cid = lax.axis_index("c")
    sid = lax.axis_index("s")
    wid = cid * NS + sid
    base = wid * (T * C)
    pltpu.sync_copy(src_hbm.at[wid], idx_v)

    def step(t, carry):
        pltpu.async_copy(x1_hbm.at[idx_v.at[t]], rows_v, sem).wait()
        pltpu.sync_copy(rows_v, out_hbm.at[pl.ds(base + t * C, C)])
        return carry

    lax.fori_loop(0, T, step, 0)


def _mlp_body(g_ref, ef_ref, w1e_ref, w2_ref, b2_ref, o_ref):
    m1 = jnp.maximum(
        g_ref[...]
        + jnp.dot(ef_ref[...], w1e_ref[...], preferred_element_type=jnp.float32),
        0.0,
    )
    o_ref[...] = jnp.maximum(
        jnp.dot(m1, w2_ref[...], preferred_element_type=jnp.float32) + b2_ref[...],
        0.0,
    )


def _mlp(g, ef_pad, w1e_t, w2_t, b2):
    BE = 5120
    grid = (E_PAD // BE,)
    return pl.pallas_call(
        _mlp_body,
        grid=grid,
        in_specs=[
            pl.BlockSpec((BE, D_OUT), lambda i: (i, 0)),
            pl.BlockSpec((BE, D_EDGE), lambda i: (i, 0)),
            pl.BlockSpec((D_EDGE, D_OUT), lambda i: (0, 0)),
            pl.BlockSpec((D_OUT, D_OUT), lambda i: (0, 0)),
            pl.BlockSpec((1, D_OUT), lambda i: (0, 0)),
        ],
        out_specs=pl.BlockSpec((BE, D_OUT), lambda i: (i, 0)),
        out_shape=jax.ShapeDtypeStruct((E_PAD, D_OUT), jnp.float32),
    )(g, ef_pad, w1e_t, w2_t, b2.reshape(1, D_OUT))


@functools.lru_cache(maxsize=None)
def _scatter_kernel():
    return pl.kernel(
        _scatter_body,
        out_type=(
            jax.ShapeDtypeStruct((NC, N_PAD, D_OUT), jnp.float32),
            jax.ShapeDtypeStruct((NC, N_PAD), jnp.float32),
        ),
        mesh=_sc_mesh(),
        scratch_types=[
            pltpu.VMEM((T, C), jnp.int32),
            pltpu.VMEM((C, D_OUT), jnp.float32),
            pltpu.VMEM((C,), jnp.float32),
            pltpu.VMEM_SHARED((N_PAD, D_OUT), jnp.float32),
            pltpu.VMEM_SHARED((N_PAD,), jnp.float32),
            pltpu.SemaphoreType.DMA,
        ],
    )


def _scatter_body(m_hbm, dst_hbm, z2_hbm, z1_hbm, s_out, d_out,
                  dst_v, mrow_v, ones_v, acc_sh, deg_sh, sem):
    cid = lax.axis_index("c")
    sid = lax.axis_index("s")
    wid = cid * NS + sid
    base = wid * (T * C)

    pltpu.sync_copy(z2_hbm.at[pl.ds(sid * NPS, NPS)],
                    acc_sh.at[pl.ds(sid * NPS, NPS)])
    pltpu.sync_copy(z1_hbm.at[pl.ds(sid * NPS, NPS)],
                    deg_sh.at[pl.ds(sid * NPS, NPS)])
    for i in range(C // 16):
        ones_v[pl.ds(i * 16, 16)] = jnp.ones((16,), jnp.float32)
    pltpu.sync_copy(dst_hbm.at[wid], dst_v)
    plsc.subcore_barrier()

    def step(t, carry):
        pltpu.sync_copy(m_hbm.at[pl.ds(base + t * C, C)], mrow_v)
        pltpu.sync_copy(mrow_v, acc_sh.at[dst_v.at[t]], add=True)
        pltpu.sync_copy(ones_v, deg_sh.at[dst_v.at[t]], add=True)
        return carry

    lax.fori_loop(0, T, step, 0)
    plsc.subcore_barrier()

    pltpu.sync_copy(acc_sh.at[pl.ds(sid * NPS, NPS)],
                    s_out.at[cid, pl.ds(sid * NPS, NPS)])
    pltpu.sync_copy(deg_sh.at[pl.ds(sid * NPS, NPS)],
                    d_out.at[cid, pl.ds(sid * NPS, NPS)])


def _apply_body(node_ref, s_ref, d_ref, wan_ref, wah_ref, b_ref, o_ref):
    s = s_ref[0] + s_ref[1]
    deg = d_ref[0] + d_ref[1]
    inv = 1.0 / jnp.maximum(deg, 1.0)
    h = s * inv[:, None]
    o_ref[...] = (
        jnp.dot(node_ref[...], wan_ref[...], preferred_element_type=jnp.float32)
        + jnp.dot(h, wah_ref[...], preferred_element_type=jnp.float32)
        + b_ref[...]
    )


def _apply(node_pad, s2, d2, wan_t, wah_t, b_apply):
    BN = 1024
    grid = (N_PAD // BN,)
    return pl.pallas_call(
        _apply_body,
        grid=grid,
        in_specs=[
            pl.BlockSpec((BN, D_NODE), lambda i: (i, 0)),
            pl.BlockSpec((NC, BN, D_OUT), lambda i: (0, i, 0)),
            pl.BlockSpec((NC, BN), lambda i: (0, i)),
            pl.BlockSpec((D_NODE, D_OUT), lambda i: (0, 0)),
            pl.BlockSpec((D_OUT, D_OUT), lambda i: (0, 0)),
            pl.BlockSpec((1, D_OUT), lambda i: (0, 0)),
        ],
        out_specs=pl.BlockSpec((BN, D_OUT), lambda i: (i, 0)),
        out_shape=jax.ShapeDtypeStruct((N_PAD, D_OUT), jnp.float32),
    )(node_pad, s2, d2, wan_t, wah_t, b_apply.reshape(1, D_OUT))


def kernel(node_features, edge_index, edge_features,
           W_msg, b_msg, W_msg_2, b_msg_2, W_apply, b_apply):
    src = edge_index[0]
    dst = edge_index[1]

    w1n_t = W_msg[:, :D_NODE].T
    w1e_t = W_msg[:, D_NODE:].T
    w2_t = W_msg_2.T
    wan_t = W_apply[:, :D_NODE].T
    wah_t = W_apply[:, D_NODE:].T

    pad = E_PAD - E
    src_p = jnp.concatenate([src, jnp.zeros((pad,), jnp.int32)])
    dst_p = jnp.concatenate([dst, jnp.full((pad,), N, jnp.int32)])
    ef_p = jnp.concatenate(
        [edge_features, jnp.zeros((pad, D_EDGE), jnp.float32)]
    )
    src_r = src_p.reshape(NW, T, C)
    dst_r = dst_p.reshape(NW, T, C)

    x1 = _x1(node_features, w1n_t, b_msg)
    g = _gather_kernel()(x1, src_r)
    m = _mlp(g, ef_p, w1e_t, w2_t, b_msg_2)
    z2 = jnp.zeros((N_PAD, D_OUT), jnp.float32)
    z1 = jnp.zeros((N_PAD,), jnp.float32)
    s2, d2 = _scatter_kernel()(m, dst_r, z2, z1)
    node_pad = jnp.pad(node_features, ((0, N_PAD - N), (0, 0)))
    out = _apply(node_pad, s2, d2, wan_t, wah_t, b_apply)
    return out[:N]


# speedup vs baseline: 2.4950x; 1.1183x over previous
import functools

import jax
import jax.numpy as jnp
from jax import lax
from jax.experimental import pallas as pl
from jax.experimental.pallas import tpu as pltpu
from jax.experimental.pallas import tpu_sc as plsc

N = 10000
E = 320000
D_NODE = 128
D_EDGE = 16
D_OUT = 128

NC = 2
NS = 16
NW = NC * NS
C = 128
E_PAD = 327680
T = E_PAD // (NW * C)
N_PAD = 10240
NPS = N_PAD // NS

@functools.lru_cache(maxsize=None)
def _sc_mesh():
    return plsc.VectorSubcoreMesh(
        core_axis_name="c", subcore_axis_name="s", num_cores=NC, num_subcores=NS
    )


def _x1_body(node_ref, w_ref, b_ref, o_ref):
    o_ref[...] = (
        jnp.dot(node_ref[...], w_ref[...], preferred_element_type=jnp.float32)
        + b_ref[...]
    )


def _x1(node, w1n_t, b_msg):
    return pl.pallas_call(
        _x1_body,
        out_shape=jax.ShapeDtypeStruct((N, D_OUT), jnp.float32),
    )(node, w1n_t, b_msg.reshape(1, D_OUT))


NBUF = 4
NG = T // NBUF
SBUF = 2
SG = T // SBUF


@functools.lru_cache(maxsize=None)
def _gather_kernel():
    return pl.kernel(
        _gather_body,
        out_type=jax.ShapeDtypeStruct((E_PAD, D_OUT), jnp.float32),
        mesh=_sc_mesh(),
        scratch_types=[
            pltpu.VMEM((T, C), jnp.int32),
            pltpu.VMEM((NBUF, C, D_OUT), jnp.float32),
            pltpu.SemaphoreType.DMA((NBUF,)),
            pltpu.SemaphoreType.DMA((NBUF,)),
        ],
    )


def _gather_body(x1_hbm, src_hbm, out_hbm, idx_v, rows_v, gsem, osem):
    cid = lax.axis_index("c")
    sid = lax.axis_index("s")
    wid = cid * NS + sid
    base = wid * (T * C)
    pltpu.sync_copy(src_hbm.at[wid], idx_v)

    for b in range(NBUF):
        pltpu.async_copy(x1_hbm.at[idx_v.at[b]], rows_v.at[b], gsem.at[b])

    def group(i, carry):
        for b in range(NBUF):
            t = i * NBUF + b
            pltpu.make_async_copy(x1_hbm.at[idx_v.at[t]],
                                  rows_v.at[b], gsem.at[b]).wait()
            pltpu.async_copy(rows_v.at[b],
                             out_hbm.at[pl.ds(base + t * C, C)], osem.at[b])
        for b in range(NBUF):
            t2 = (i + 1) * NBUF + b
            pltpu.make_async_copy(rows_v.at[b],
                                  out_hbm.at[pl.ds(base, C)], osem.at[b]).wait()
            pltpu.async_copy(x1_hbm.at[idx_v.at[t2]], rows_v.at[b], gsem.at[b])
        return carry

    lax.fori_loop(0, NG - 1, group, 0)

    for b in range(NBUF):
        t = (NG - 1) * NBUF + b
        pltpu.make_async_copy(x1_hbm.at[idx_v.at[t]],
                              rows_v.at[b], gsem.at[b]).wait()
        pltpu.async_copy(rows_v.at[b],
                         out_hbm.at[pl.ds(base + t * C, C)], osem.at[b])
    for b in range(NBUF):
        pltpu.make_async_copy(rows_v.at[b],
                              out_hbm.at[pl.ds(base, C)], osem.at[b]).wait()


def _mlp_body(g_ref, ef_ref, w1e_ref, w2_ref, b2_ref, o_ref):
    m1 = jnp.maximum(
        g_ref[...]
        + jnp.dot(ef_ref[...], w1e_ref[...], preferred_element_type=jnp.float32),
        0.0,
    )
    o_ref[...] = jnp.maximum(
        jnp.dot(m1, w2_ref[...], preferred_element_type=jnp.float32) + b2_ref[...],
        0.0,
    )


def _mlp(g, ef_pad, w1e_t, w2_t, b2):
    BE = 5120
    grid = (E_PAD // BE,)
    return pl.pallas_call(
        _mlp_body,
        grid=grid,
        in_specs=[
            pl.BlockSpec((BE, D_OUT), lambda i: (i, 0)),
            pl.BlockSpec((BE, D_EDGE), lambda i: (i, 0)),
            pl.BlockSpec((D_EDGE, D_OUT), lambda i: (0, 0)),
            pl.BlockSpec((D_OUT, D_OUT), lambda i: (0, 0)),
            pl.BlockSpec((1, D_OUT), lambda i: (0, 0)),
        ],
        out_specs=pl.BlockSpec((BE, D_OUT), lambda i: (i, 0)),
        out_shape=jax.ShapeDtypeStruct((E_PAD, D_OUT), jnp.float32),
    )(g, ef_pad, w1e_t, w2_t, b2.reshape(1, D_OUT))


@functools.lru_cache(maxsize=None)
def _scatter_kernel():
    return pl.kernel(
        _scatter_body,
        out_type=(
            jax.ShapeDtypeStruct((NC, N_PAD, D_OUT), jnp.float32),
            jax.ShapeDtypeStruct((NC, N_PAD), jnp.float32),
        ),
        mesh=_sc_mesh(),
        scratch_types=[
            pltpu.VMEM((T, C), jnp.int32),
            pltpu.VMEM((SBUF, C, D_OUT), jnp.float32),
            pltpu.VMEM((C,), jnp.float32),
            pltpu.VMEM_SHARED((N_PAD, D_OUT), jnp.float32),
            pltpu.VMEM_SHARED((N_PAD,), jnp.float32),
            pltpu.SemaphoreType.DMA((SBUF,)),
            pltpu.SemaphoreType.DMA((SBUF,)),
            pltpu.SemaphoreType.DMA((SBUF,)),
        ],
    )


def _scatter_body(m_hbm, dst_hbm, z2_hbm, z1_hbm, s_out, d_out,
                  dst_v, mrow_v, ones_v, acc_sh, deg_sh, lsem, asem, dsem):
    cid = lax.axis_index("c")
    sid = lax.axis_index("s")
    wid = cid * NS + sid
    base = wid * (T * C)

    pltpu.sync_copy(z2_hbm.at[pl.ds(sid * NPS, NPS)],
                    acc_sh.at[pl.ds(sid * NPS, NPS)])
    pltpu.sync_copy(z1_hbm.at[pl.ds(sid * NPS, NPS)],
                    deg_sh.at[pl.ds(sid * NPS, NPS)])
    for i in range(C // 16):
        ones_v[pl.ds(i * 16, 16)] = jnp.ones((16,), jnp.float32)
    pltpu.sync_copy(dst_hbm.at[wid], dst_v)
    plsc.subcore_barrier()

    for b in range(SBUF):
        pltpu.async_copy(m_hbm.at[pl.ds(base + b * C, C)],
                         mrow_v.at[b], lsem.at[b])

    def group(i, carry):
        for b in range(SBUF):
            t = i * SBUF + b
            pltpu.make_async_copy(m_hbm.at[pl.ds(base, C)],
                                  mrow_v.at[b], lsem.at[b]).wait()
            pltpu.async_copy(mrow_v.at[b], acc_sh.at[dst_v.at[t]],
                             asem.at[b], add=True)
            pltpu.async_copy(ones_v, deg_sh.at[dst_v.at[t]],
                             dsem.at[b], add=True)
        for b in range(SBUF):
            t2 = (i + 1) * SBUF + b
            pltpu.make_async_copy(mrow_v.at[b], acc_sh.at[dst_v.at[0]],
                                  asem.at[b]).wait()
            pltpu.make_async_copy(ones_v, deg_sh.at[dst_v.at[0]],
                                  dsem.at[b]).wait()
            pltpu.async_copy(m_hbm.at[pl.ds(base + t2 * C, C)],
                             mrow_v.at[b], lsem.at[b])
        return carry

    lax.fori_loop(0, SG - 1, group, 0)

    for b in range(SBUF):
        t = (SG - 1) * SBUF + b
        pltpu.make_async_copy(m_hbm.at[pl.ds(base, C)],
                              mrow_v.at[b], lsem.at[b]).wait()
        pltpu.async_copy(mrow_v.at[b], acc_sh.at[dst_v.at[t]],
                         asem.at[b], add=True)
        pltpu.async_copy(ones_v, deg_sh.at[dst_v.at[t]],
                         dsem.at[b], add=True)
    for b in range(SBUF):
        pltpu.make_async_copy(mrow_v.at[b], acc_sh.at[dst_v.at[0]],
                              asem.at[b]).wait()
        pltpu.make_async_copy(ones_v, deg_sh.at[dst_v.at[0]],
                              dsem.at[b]).wait()
    plsc.subcore_barrier()

    pltpu.sync_copy(acc_sh.at[pl.ds(sid * NPS, NPS)],
                    s_out.at[cid, pl.ds(sid * NPS, NPS)])
    pltpu.sync_copy(deg_sh.at[pl.ds(sid * NPS, NPS)],
                    d_out.at[cid, pl.ds(sid * NPS, NPS)])


def _apply_body(node_ref, s_ref, d_ref, wan_ref, wah_ref, b_ref, o_ref):
    s = s_ref[0] + s_ref[1]
    deg = d_ref[0] + d_ref[1]
    inv = 1.0 / jnp.maximum(deg, 1.0)
    h = s * inv[:, None]
    o_ref[...] = (
        jnp.dot(node_ref[...], wan_ref[...], preferred_element_type=jnp.float32)
        + jnp.dot(h, wah_ref[...], preferred_element_type=jnp.float32)
        + b_ref[...]
    )


def _apply(node_pad, s2, d2, wan_t, wah_t, b_apply):
    BN = 1024
    grid = (N_PAD // BN,)
    return pl.pallas_call(
        _apply_body,
        grid=grid,
        in_specs=[
            pl.BlockSpec((BN, D_NODE), lambda i: (i, 0)),
            pl.BlockSpec((NC, BN, D_OUT), lambda i: (0, i, 0)),
            pl.BlockSpec((NC, BN), lambda i: (0, i)),
            pl.BlockSpec((D_NODE, D_OUT), lambda i: (0, 0)),
            pl.BlockSpec((D_OUT, D_OUT), lambda i: (0, 0)),
            pl.BlockSpec((1, D_OUT), lambda i: (0, 0)),
        ],
        out_specs=pl.BlockSpec((BN, D_OUT), lambda i: (i, 0)),
        out_shape=jax.ShapeDtypeStruct((N_PAD, D_OUT), jnp.float32),
    )(node_pad, s2, d2, wan_t, wah_t, b_apply.reshape(1, D_OUT))


def kernel(node_features, edge_index, edge_features,
           W_msg, b_msg, W_msg_2, b_msg_2, W_apply, b_apply):
    src = edge_index[0]
    dst = edge_index[1]

    w1n_t = W_msg[:, :D_NODE].T
    w1e_t = W_msg[:, D_NODE:].T
    w2_t = W_msg_2.T
    wan_t = W_apply[:, :D_NODE].T
    wah_t = W_apply[:, D_NODE:].T

    pad = E_PAD - E
    src_p = jnp.concatenate([src, jnp.zeros((pad,), jnp.int32)])
    dst_p = jnp.concatenate([dst, jnp.full((pad,), N, jnp.int32)])
    ef_p = jnp.concatenate(
        [edge_features, jnp.zeros((pad, D_EDGE), jnp.float32)]
    )
    src_r = src_p.reshape(NW, T, C)
    dst_r = dst_p.reshape(NW, T, C)

    x1 = _x1(node_features, w1n_t, b_msg)
    g = _gather_kernel()(x1, src_r)
    m = _mlp(g, ef_p, w1e_t, w2_t, b_msg_2)
    z2 = jnp.zeros((N_PAD, D_OUT), jnp.float32)
    z1 = jnp.zeros((N_PAD,), jnp.float32)
    s2, d2 = _scatter_kernel()(m, dst_r, z2, z1)
    node_pad = jnp.pad(node_features, ((0, N_PAD - N), (0, 0)))
    out = _apply(node_pad, s2, d2, wan_t, wah_t, b_apply)
    return out[:N]


# speedup vs baseline: 4.7325x; 1.8968x over previous
import functools

import jax
import jax.numpy as jnp
from jax import lax
from jax.experimental import pallas as pl
from jax.experimental.pallas import tpu as pltpu
from jax.experimental.pallas import tpu_sc as plsc

N = 10000
E = 320000
D_NODE = 128
D_EDGE = 16
D_OUT = 128

NC = 2
NS = 16
NW = NC * NS
C = 128
E_PAD = 327680
T = E_PAD // (NW * C)
N_PAD = 10240
NPS = N_PAD // NS

@functools.lru_cache(maxsize=None)
def _sc_mesh():
    return plsc.VectorSubcoreMesh(
        core_axis_name="c", subcore_axis_name="s", num_cores=NC, num_subcores=NS
    )


def _x1_body(node_ref, w_ref, b_ref, o_ref):
    o_ref[...] = (
        jnp.dot(node_ref[...], w_ref[...], preferred_element_type=jnp.float32)
        + b_ref[...]
    )


def _x1(node_pad, w1n_t, b_msg):
    return pl.pallas_call(
        _x1_body,
        out_shape=jax.ShapeDtypeStruct((N_PAD, D_OUT), jnp.float32),
    )(node_pad, w1n_t, b_msg.reshape(1, D_OUT))


NBUF = 4
NG = T // NBUF
SBUF = 2
SG = T // SBUF


GBUF = 2


@functools.lru_cache(maxsize=None)
def _gather_kernel():
    return pl.kernel(
        _gather_body,
        out_type=jax.ShapeDtypeStruct((E_PAD, D_OUT), jnp.float32),
        mesh=_sc_mesh(),
        scratch_types=[
            pltpu.VMEM((T, C), jnp.int32),
            pltpu.VMEM((GBUF, C, D_OUT), jnp.float32),
            pltpu.VMEM_SHARED((N_PAD, D_OUT), jnp.float32),
            pltpu.SemaphoreType.DMA((GBUF,)),
            pltpu.SemaphoreType.DMA((GBUF,)),
        ],
    )


def _gather_body(x1_hbm, src_hbm, out_hbm, idx_v, rows_v, x1_sh, gsem, osem):
    cid = lax.axis_index("c")
    sid = lax.axis_index("s")
    wid = cid * NS + sid
    base = wid * (T * C)
    pltpu.sync_copy(x1_hbm.at[pl.ds(sid * NPS, NPS)],
                    x1_sh.at[pl.ds(sid * NPS, NPS)])
    pltpu.sync_copy(src_hbm.at[wid], idx_v)
    plsc.subcore_barrier()

    for b in range(GBUF):
        pltpu.async_copy(x1_sh.at[idx_v.at[b]], rows_v.at[b], gsem.at[b])

    def group(i, carry):
        for b in range(GBUF):
            t = i * GBUF + b
            pltpu.make_async_copy(x1_sh.at[idx_v.at[t]],
                                  rows_v.at[b], gsem.at[b]).wait()
            pltpu.async_copy(rows_v.at[b],
                             out_hbm.at[pl.ds(base + t * C, C)], osem.at[b])
        for b in range(GBUF):
            t2 = (i + 1) * GBUF + b
            pltpu.make_async_copy(rows_v.at[b],
                                  out_hbm.at[pl.ds(base, C)], osem.at[b]).wait()
            pltpu.async_copy(x1_sh.at[idx_v.at[t2]], rows_v.at[b], gsem.at[b])
        return carry

    lax.fori_loop(0, T // GBUF - 1, group, 0)

    for b in range(GBUF):
        t = (T // GBUF - 1) * GBUF + b
        pltpu.make_async_copy(x1_sh.at[idx_v.at[t]],
                              rows_v.at[b], gsem.at[b]).wait()
        pltpu.async_copy(rows_v.at[b],
                         out_hbm.at[pl.ds(base + t * C, C)], osem.at[b])
    for b in range(GBUF):
        pltpu.make_async_copy(rows_v.at[b],
                              out_hbm.at[pl.ds(base, C)], osem.at[b]).wait()


def _mlp_body(g_ref, ef_ref, w1e_ref, w2_ref, b2_ref, o_ref):
    m1 = jnp.maximum(
        g_ref[...]
        + jnp.dot(ef_ref[...], w1e_ref[...], preferred_element_type=jnp.float32),
        0.0,
    )
    o_ref[...] = jnp.maximum(
        jnp.dot(m1, w2_ref[...], preferred_element_type=jnp.float32) + b2_ref[...],
        0.0,
    )


def _mlp(g, ef_pad, w1e_t, w2_t, b2):
    BE = 5120
    grid = (E_PAD // BE,)
    return pl.pallas_call(
        _mlp_body,
        grid=grid,
        in_specs=[
            pl.BlockSpec((BE, D_OUT), lambda i: (i, 0)),
            pl.BlockSpec((BE, D_EDGE), lambda i: (i, 0)),
            pl.BlockSpec((D_EDGE, D_OUT), lambda i: (0, 0)),
            pl.BlockSpec((D_OUT, D_OUT), lambda i: (0, 0)),
            pl.BlockSpec((1, D_OUT), lambda i: (0, 0)),
        ],
        out_specs=pl.BlockSpec((BE, D_OUT), lambda i: (i, 0)),
        out_shape=jax.ShapeDtypeStruct((E_PAD, D_OUT), jnp.float32),
    )(g, ef_pad, w1e_t, w2_t, b2.reshape(1, D_OUT))


@functools.lru_cache(maxsize=None)
def _scatter_kernel():
    return pl.kernel(
        _scatter_body,
        out_type=(
            jax.ShapeDtypeStruct((NC, N_PAD, D_OUT), jnp.float32),
            jax.ShapeDtypeStruct((NC, N_PAD), jnp.float32),
        ),
        mesh=_sc_mesh(),
        scratch_types=[
            pltpu.VMEM((T, C), jnp.int32),
            pltpu.VMEM((SBUF, C, D_OUT), jnp.float32),
            pltpu.VMEM((C,), jnp.float32),
            pltpu.VMEM_SHARED((N_PAD, D_OUT), jnp.float32),
            pltpu.VMEM_SHARED((N_PAD,), jnp.float32),
            pltpu.SemaphoreType.DMA((SBUF,)),
            pltpu.SemaphoreType.DMA((SBUF,)),
            pltpu.SemaphoreType.DMA((SBUF,)),
        ],
    )


def _scatter_body(m_hbm, dst_hbm, z2_hbm, z1_hbm, s_out, d_out,
                  dst_v, mrow_v, ones_v, acc_sh, deg_sh, lsem, asem, dsem):
    cid = lax.axis_index("c")
    sid = lax.axis_index("s")
    wid = cid * NS + sid
    base = wid * (T * C)

    pltpu.sync_copy(z2_hbm.at[pl.ds(sid * NPS, NPS)],
                    acc_sh.at[pl.ds(sid * NPS, NPS)])
    pltpu.sync_copy(z1_hbm.at[pl.ds(sid * NPS, NPS)],
                    deg_sh.at[pl.ds(sid * NPS, NPS)])
    for i in range(C // 16):
        ones_v[pl.ds(i * 16, 16)] = jnp.ones((16,), jnp.float32)
    pltpu.sync_copy(dst_hbm.at[wid], dst_v)
    plsc.subcore_barrier()

    for b in range(SBUF):
        pltpu.async_copy(m_hbm.at[pl.ds(base + b * C, C)],
                         mrow_v.at[b], lsem.at[b])

    def group(i, carry):
        for b in range(SBUF):
            t = i * SBUF + b
            pltpu.make_async_copy(m_hbm.at[pl.ds(base, C)],
                                  mrow_v.at[b], lsem.at[b]).wait()
            pltpu.async_copy(mrow_v.at[b], acc_sh.at[dst_v.at[t]],
                             asem.at[b], add=True)
            pltpu.async_copy(ones_v, deg_sh.at[dst_v.at[t]],
                             dsem.at[b], add=True)
        for b in range(SBUF):
            t2 = (i + 1) * SBUF + b
            pltpu.make_async_copy(mrow_v.at[b], acc_sh.at[dst_v.at[0]],
                                  asem.at[b]).wait()
            pltpu.make_async_copy(ones_v, deg_sh.at[dst_v.at[0]],
                                  dsem.at[b]).wait()
            pltpu.async_copy(m_hbm.at[pl.ds(base + t2 * C, C)],
                             mrow_v.at[b], lsem.at[b])
        return carry

    lax.fori_loop(0, SG - 1, group, 0)

    for b in range(SBUF):
        t = (SG - 1) * SBUF + b
        pltpu.make_async_copy(m_hbm.at[pl.ds(base, C)],
                              mrow_v.at[b], lsem.at[b]).wait()
        pltpu.async_copy(mrow_v.at[b], acc_sh.at[dst_v.at[t]],
                         asem.at[b], add=True)
        pltpu.async_copy(ones_v, deg_sh.at[dst_v.at[t]],
                         dsem.at[b], add=True)
    for b in range(SBUF):
        pltpu.make_async_copy(mrow_v.at[b], acc_sh.at[dst_v.at[0]],
                              asem.at[b]).wait()
        pltpu.make_async_copy(ones_v, deg_sh.at[dst_v.at[0]],
                              dsem.at[b]).wait()
    plsc.subcore_barrier()

    pltpu.sync_copy(acc_sh.at[pl.ds(sid * NPS, NPS)],
                    s_out.at[cid, pl.ds(sid * NPS, NPS)])
    pltpu.sync_copy(deg_sh.at[pl.ds(sid * NPS, NPS)],
                    d_out.at[cid, pl.ds(sid * NPS, NPS)])


def _apply_body(node_ref, s_ref, d_ref, wan_ref, wah_ref, b_ref, o_ref):
    s = s_ref[0] + s_ref[1]
    deg = d_ref[0] + d_ref[1]
    inv = 1.0 / jnp.maximum(deg, 1.0)
    h = s * inv[:, None]
    o_ref[...] = (
        jnp.dot(node_ref[...], wan_ref[...], preferred_element_type=jnp.float32)
        + jnp.dot(h, wah_ref[...], preferred_element_type=jnp.float32)
        + b_ref[...]
    )


def _apply(node_pad, s2, d2, wan_t, wah_t, b_apply):
    BN = 1024
    grid = (N_PAD // BN,)
    return pl.pallas_call(
        _apply_body,
        grid=grid,
        in_specs=[
            pl.BlockSpec((BN, D_NODE), lambda i: (i, 0)),
            pl.BlockSpec((NC, BN, D_OUT), lambda i: (0, i, 0)),
            pl.BlockSpec((NC, BN), lambda i: (0, i)),
            pl.BlockSpec((D_NODE, D_OUT), lambda i: (0, 0)),
            pl.BlockSpec((D_OUT, D_OUT), lambda i: (0, 0)),
            pl.BlockSpec((1, D_OUT), lambda i: (0, 0)),
        ],
        out_specs=pl.BlockSpec((BN, D_OUT), lambda i: (i, 0)),
        out_shape=jax.ShapeDtypeStruct((N_PAD, D_OUT), jnp.float32),
    )(node_pad, s2, d2, wan_t, wah_t, b_apply.reshape(1, D_OUT))


def kernel(node_features, edge_index, edge_features,
           W_msg, b_msg, W_msg_2, b_msg_2, W_apply, b_apply):
    src = edge_index[0]
    dst = edge_index[1]

    w1n_t = W_msg[:, :D_NODE].T
    w1e_t = W_msg[:, D_NODE:].T
    w2_t = W_msg_2.T
    wan_t = W_apply[:, :D_NODE].T
    wah_t = W_apply[:, D_NODE:].T

    pad = E_PAD - E
    src_p = jnp.concatenate([src, jnp.zeros((pad,), jnp.int32)])
    dst_p = jnp.concatenate([dst, jnp.full((pad,), N, jnp.int32)])
    ef_p = jnp.concatenate(
        [edge_features, jnp.zeros((pad, D_EDGE), jnp.float32)]
    )
    src_r = src_p.reshape(NW, T, C)
    dst_r = dst_p.reshape(NW, T, C)

    node_pad = jnp.pad(node_features, ((0, N_PAD - N), (0, 0)))
    x1 = _x1(node_pad, w1n_t, b_msg)
    g = _gather_kernel()(x1, src_r)
    m = _mlp(g, ef_p, w1e_t, w2_t, b_msg_2)
    z2 = jnp.zeros((N_PAD, D_OUT), jnp.float32)
    z1 = jnp.zeros((N_PAD,), jnp.float32)
    s2, d2 = _scatter_kernel()(m, dst_r, z2, z1)
    out = _apply(node_pad, s2, d2, wan_t, wah_t, b_apply)
    return out[:N]


# speedup vs baseline: 5.4188x; 1.1450x over previous
import functools

import jax
import jax.numpy as jnp
from jax import lax
from jax.experimental import pallas as pl
from jax.experimental.pallas import tpu as pltpu
from jax.experimental.pallas import tpu_sc as plsc

N = 10000
E = 320000
D_NODE = 128
D_EDGE = 16
D_OUT = 128

NC = 2
NS = 16
NW = NC * NS
C = 128
E_PAD = 327680
T = E_PAD // (NW * C)
N_PAD = 10240
NPS = N_PAD // NS

KSPLIT = 2
E_SPLIT = E_PAD // KSPLIT
TK = T // KSPLIT

GBUF = 2
SBUF = 2

BE = 5120
BPS = E_SPLIT // BE
EF_LAST = (E - 1) // BE


@functools.lru_cache(maxsize=None)
def _sc_mesh():
    return plsc.VectorSubcoreMesh(
        core_axis_name="c", subcore_axis_name="s", num_cores=NC, num_subcores=NS
    )


def _x1_body(node_ref, w_ref, b_ref, o_ref):
    o_ref[...] = (
        jnp.dot(node_ref[...], w_ref[...], preferred_element_type=jnp.float32)
        + b_ref[...]
    )


def _x1(node_pad, w1n_t, b_msg):
    return pl.pallas_call(
        _x1_body,
        out_shape=jax.ShapeDtypeStruct((N_PAD, D_OUT), jnp.float32),
    )(node_pad, w1n_t, b_msg.reshape(1, D_OUT))


def _gather_body(x1_hbm, src_hbm, out_hbm, idx_v, rows_v, x1_sh, gsem, osem):
    cid = lax.axis_index("c")
    sid = lax.axis_index("s")
    wid = cid * NS + sid
    base = wid * (TK * C)
    pltpu.sync_copy(x1_hbm.at[pl.ds(sid * NPS, NPS)],
                    x1_sh.at[pl.ds(sid * NPS, NPS)])
    pltpu.sync_copy(src_hbm.at[wid], idx_v)
    plsc.subcore_barrier()

    for b in range(GBUF):
        pltpu.async_copy(x1_sh.at[idx_v.at[b]], rows_v.at[b], gsem.at[b])

    def group(i, carry):
        for b in range(GBUF):
            t = i * GBUF + b
            pltpu.make_async_copy(x1_sh.at[idx_v.at[t]],
                                  rows_v.at[b], gsem.at[b]).wait()
            pltpu.async_copy(rows_v.at[b],
                             out_hbm.at[pl.ds(base + t * C, C)], osem.at[b])
        for b in range(GBUF):
            t2 = (i + 1) * GBUF + b
            pltpu.make_async_copy(rows_v.at[b],
                                  out_hbm.at[pl.ds(base, C)], osem.at[b]).wait()
            pltpu.async_copy(x1_sh.at[idx_v.at[t2]], rows_v.at[b], gsem.at[b])
        return carry

    lax.fori_loop(0, TK // GBUF - 1, group, 0)

    for b in range(GBUF):
        t = (TK // GBUF - 1) * GBUF + b
        pltpu.make_async_copy(x1_sh.at[idx_v.at[t]],
                              rows_v.at[b], gsem.at[b]).wait()
        pltpu.async_copy(rows_v.at[b],
                         out_hbm.at[pl.ds(base + t * C, C)], osem.at[b])
    for b in range(GBUF):
        pltpu.make_async_copy(rows_v.at[b],
                              out_hbm.at[pl.ds(base, C)], osem.at[b]).wait()


@functools.lru_cache(maxsize=None)
def _gather_kernel():
    return pl.kernel(
        _gather_body,
        out_type=jax.ShapeDtypeStruct((E_SPLIT, D_OUT), jnp.float32),
        mesh=_sc_mesh(),
        scratch_types=[
            pltpu.VMEM((TK, C), jnp.int32),
            pltpu.VMEM((GBUF, C, D_OUT), jnp.float32),
            pltpu.VMEM_SHARED((N_PAD, D_OUT), jnp.float32),
            pltpu.SemaphoreType.DMA((GBUF,)),
            pltpu.SemaphoreType.DMA((GBUF,)),
        ],
    )


def _mlp_body(g_ref, ef_ref, w1e_ref, w2_ref, b2_ref, o_ref):
    m1 = jnp.maximum(
        g_ref[...]
        + jnp.dot(ef_ref[...], w1e_ref[...], preferred_element_type=jnp.float32),
        0.0,
    )
    o_ref[...] = jnp.maximum(
        jnp.dot(m1, w2_ref[...], preferred_element_type=jnp.float32) + b2_ref[...],
        0.0,
    )


def _mlp(g, ef, w1e_t, w2_t, b2, split):
    off = split * BPS

    def ef_map(i):
        return (jnp.minimum(off + i, EF_LAST), 0)

    return pl.pallas_call(
        _mlp_body,
        grid=(BPS,),
        in_specs=[
            pl.BlockSpec((BE, D_OUT), lambda i: (i, 0)),
            pl.BlockSpec((BE, D_EDGE), ef_map),
            pl.BlockSpec((D_EDGE, D_OUT), lambda i: (0, 0)),
            pl.BlockSpec((D_OUT, D_OUT), lambda i: (0, 0)),
            pl.BlockSpec((1, D_OUT), lambda i: (0, 0)),
        ],
        out_specs=pl.BlockSpec((BE, D_OUT), lambda i: (i, 0)),
        out_shape=jax.ShapeDtypeStruct((E_SPLIT, D_OUT), jnp.float32),
    )(g, ef, w1e_t, w2_t, b2.reshape(1, D_OUT))


def _scatter_body(m_hbm, dst_hbm, p2_hbm, p1_hbm, s_out, d_out,
                  dst_v, mrow_v, ones_v, acc_sh, deg_sh, lsem, asem, dsem):
    cid = lax.axis_index("c")
    sid = lax.axis_index("s")
    wid = cid * NS + sid
    base = wid * (TK * C)

    pltpu.sync_copy(p2_hbm.at[cid, pl.ds(sid * NPS, NPS)],
                    acc_sh.at[pl.ds(sid * NPS, NPS)])
    pltpu.sync_copy(p1_hbm.at[cid, pl.ds(sid * NPS, NPS)],
                    deg_sh.at[pl.ds(sid * NPS, NPS)])
    for i in range(C // 16):
        ones_v[pl.ds(i * 16, 16)] = jnp.ones((16,), jnp.float32)
    pltpu.sync_copy(dst_hbm.at[wid], dst_v)
    plsc.subcore_barrier()

    for b in range(SBUF):
        pltpu.async_copy(m_hbm.at[pl.ds(base + b * C, C)],
                         mrow_v.at[b], lsem.at[b])

    def group(i, carry):
        for b in range(SBUF):
            t = i * SBUF + b
            pltpu.make_async_copy(m_hbm.at[pl.ds(base, C)],
                                  mrow_v.at[b], lsem.at[b]).wait()
            pltpu.async_copy(mrow_v.at[b], acc_sh.at[dst_v.at[t]],
                             asem.at[b], add=True)
            pltpu.async_copy(ones_v, deg_sh.at[dst_v.at[t]],
                             dsem.at[b], add=True)
        for b in range(SBUF):
            t2 = (i + 1) * SBUF + b
            pltpu.make_async_copy(mrow_v.at[b], acc_sh.at[dst_v.at[0]],
                                  asem.at[b]).wait()
            pltpu.make_async_copy(ones_v, deg_sh.at[dst_v.at[0]],
                                  dsem.at[b]).wait()
            pltpu.async_copy(m_hbm.at[pl.ds(base + t2 * C, C)],
                             mrow_v.at[b], lsem.at[b])
        return carry

    lax.fori_loop(0, TK // SBUF - 1, group, 0)

    for b in range(SBUF):
        t = (TK // SBUF - 1) * SBUF + b
        pltpu.make_async_copy(m_hbm.at[pl.ds(base, C)],
                              mrow_v.at[b], lsem.at[b]).wait()
        pltpu.async_copy(mrow_v.at[b], acc_sh.at[dst_v.at[t]],
                         asem.at[b], add=True)
        pltpu.async_copy(ones_v, deg_sh.at[dst_v.at[t]],
                         dsem.at[b], add=True)
    for b in range(SBUF):
        pltpu.make_async_copy(mrow_v.at[b], acc_sh.at[dst_v.at[0]],
                              asem.at[b]).wait()
        pltpu.make_async_copy(ones_v, deg_sh.at[dst_v.at[0]],
                              dsem.at[b]).wait()
    plsc.subcore_barrier()

    pltpu.sync_copy(acc_sh.at[pl.ds(sid * NPS, NPS)],
                    s_out.at[cid, pl.ds(sid * NPS, NPS)])
    pltpu.sync_copy(deg_sh.at[pl.ds(sid * NPS, NPS)],
                    d_out.at[cid, pl.ds(sid * NPS, NPS)])


@functools.lru_cache(maxsize=None)
def _scatter_kernel():
    return pl.kernel(
        _scatter_body,
        out_type=(
            jax.ShapeDtypeStruct((NC, N_PAD, D_OUT), jnp.float32),
            jax.ShapeDtypeStruct((NC, N_PAD), jnp.float32),
        ),
        mesh=_sc_mesh(),
        scratch_types=[
            pltpu.VMEM((TK, C), jnp.int32),
            pltpu.VMEM((SBUF, C, D_OUT), jnp.float32),
            pltpu.VMEM((C,), jnp.float32),
            pltpu.VMEM_SHARED((N_PAD, D_OUT), jnp.float32),
            pltpu.VMEM_SHARED((N_PAD,), jnp.float32),
            pltpu.SemaphoreType.DMA((SBUF,)),
            pltpu.SemaphoreType.DMA((SBUF,)),
            pltpu.SemaphoreType.DMA((SBUF,)),
        ],
    )


def _apply_body(node_ref, s_ref, d_ref, wan_ref, wah_ref, b_ref, o_ref):
    s = s_ref[0] + s_ref[1]
    deg = d_ref[0] + d_ref[1]
    inv = 1.0 / jnp.maximum(deg, 1.0)
    h = s * inv[:, None]
    o_ref[...] = (
        jnp.dot(node_ref[...], wan_ref[...], preferred_element_type=jnp.float32)
        + jnp.dot(h, wah_ref[...], preferred_element_type=jnp.float32)
        + b_ref[...]
    )


def _apply(node_pad, s2, d2, wan_t, wah_t, b_apply):
    BN = 1024
    grid = (N_PAD // BN,)
    return pl.pallas_call(
        _apply_body,
        grid=grid,
        in_specs=[
            pl.BlockSpec((BN, D_NODE), lambda i: (i, 0)),
            pl.BlockSpec((NC, BN, D_OUT), lambda i: (0, i, 0)),
            pl.BlockSpec((NC, BN), lambda i: (0, i)),
            pl.BlockSpec((D_NODE, D_OUT), lambda i: (0, 0)),
            pl.BlockSpec((D_OUT, D_OUT), lambda i: (0, 0)),
            pl.BlockSpec((1, D_OUT), lambda i: (0, 0)),
        ],
        out_specs=pl.BlockSpec((BN, D_OUT), lambda i: (i, 0)),
        out_shape=jax.ShapeDtypeStruct((N_PAD, D_OUT), jnp.float32),
    )(node_pad, s2, d2, wan_t, wah_t, b_apply.reshape(1, D_OUT))


def kernel(node_features, edge_index, edge_features,
           W_msg, b_msg, W_msg_2, b_msg_2, W_apply, b_apply):
    src = edge_index[0]
    dst = edge_index[1]

    w1n_t = W_msg[:, :D_NODE].T
    w1e_t = W_msg[:, D_NODE:].T
    w2_t = W_msg_2.T
    wan_t = W_apply[:, :D_NODE].T
    wah_t = W_apply[:, D_NODE:].T

    pad = E_PAD - E
    src_p = jnp.concatenate([src, jnp.zeros((pad,), jnp.int32)])
    dst_p = jnp.concatenate([dst, jnp.full((pad,), N, jnp.int32)])
    src_r = src_p.reshape(KSPLIT, NW, TK, C)
    dst_r = dst_p.reshape(KSPLIT, NW, TK, C)

    node_pad = jnp.pad(node_features, ((0, N_PAD - N), (0, 0)))
    x1 = _x1(node_pad, w1n_t, b_msg)

    s2 = jnp.zeros((NC, N_PAD, D_OUT), jnp.float32)
    d2 = jnp.zeros((NC, N_PAD), jnp.float32)
    for k in range(KSPLIT):
        g = _gather_kernel()(x1, src_r[k])
        m = _mlp(g, edge_features, w1e_t, w2_t, b_msg_2, k)
        s2, d2 = _scatter_kernel()(m, dst_r[k], s2, d2)

    out = _apply(node_pad, s2, d2, wan_t, wah_t, b_apply)
    return out[:N]
